# Initial kernel scaffold; baseline (speedup 1.0000x reference)
import jax
import jax.numpy as jnp
from jax.experimental import pallas as pl


def kernel(x_dict, edge_index, edge_type, node_type, local_node_idx, W_rel, W_root, b_root):
    raise NotImplementedError("write your pallas kernel here")


# speedup vs baseline: 3.9160x; 3.9160x over previous
import functools

import jax
import jax.numpy as jnp
from jax import lax
from jax.experimental import pallas as pl
from jax.experimental.pallas import tpu as pltpu
from jax.experimental.pallas import tpu_sc as plsc

_NC = 2
_NS = 16
_CW = 32
_BB = 128
_NBI = 8
_SB = _BB * _NBI


def _round_up(a, b):
    return (a + b - 1) // b * b


@functools.lru_cache(maxsize=None)
def _make_sc_segsum(N, E_pad, NCH, NKEY, NKEYP, with_counts):
    ZPT = NKEYP // _NS
    EPT = E_pad // _NS
    EPC = E_pad // (_NS * _NC)
    CPC = NCH // _NC
    nsb_data = EPT // _SB
    nsb_cnt = EPC // _SB
    EROWS = E_pad // _BB
    f32 = jnp.float32

    mesh = plsc.VectorSubcoreMesh(core_axis_name="c", subcore_axis_name="s")

    def body(xflat, src8, key2, zeros_h, ones_h, *refs):
        if with_counts:
            s_out, c_out, acc, kbuf, ibuf, dbuf, obuf, sem = refs
        else:
            s_out, acc, kbuf, ibuf, dbuf, obuf, sem = refs
        cid = lax.axis_index("c")
        sid = lax.axis_index("s")
        pltpu.sync_copy(ones_h, obuf)

        def zero_acc():
            pltpu.sync_copy(zeros_h, acc.at[pl.ds(sid * ZPT, ZPT)])

        def run_pass(row_base, nsb, chunk):
            def super_step(sb, carry):
                rk = row_base + sb * _NBI
                pltpu.sync_copy(key2.at[pl.ds(rk, _NBI)], kbuf)
                if chunk is not None:
                    ri = chunk * EROWS + rk
                    pltpu.sync_copy(src8.at[pl.ds(ri, _NBI)], ibuf)
                    cps = [pltpu.async_copy(xflat.at[ibuf.at[j]], dbuf.at[j], sem)
                           for j in range(_NBI)]
                    for cp in cps:
                        cp.wait()
                    for j in range(_NBI):
                        pltpu.sync_copy(dbuf.at[j], acc.at[kbuf.at[j]], add=True)
                else:
                    for j in range(_NBI):
                        pltpu.sync_copy(obuf, acc.at[kbuf.at[j]], add=True)
                return carry
            lax.fori_loop(0, nsb, super_step, 0)

        for p in range(CPC):
            chunk = cid * CPC + p
            zero_acc()
            plsc.subcore_barrier()
            run_pass(sid * (EPT // _BB), nsb_data, chunk)
            plsc.subcore_barrier()
            pltpu.sync_copy(acc.at[pl.ds(sid * ZPT, ZPT)],
                            s_out.at[pl.ds(chunk * NKEYP + sid * ZPT, ZPT)])
            plsc.subcore_barrier()

        if with_counts:
            zero_acc()
            plsc.subcore_barrier()
            run_pass(cid * (EPC * _NS // _BB) + sid * (EPC // _BB), nsb_cnt, None)
            plsc.subcore_barrier()
            pltpu.sync_copy(acc.at[pl.ds(sid * ZPT, ZPT)],
                            c_out.at[pl.ds(cid * NKEYP + sid * ZPT, ZPT)])

    out_type = [jax.ShapeDtypeStruct((NCH * NKEYP, _CW), f32)]
    if with_counts:
        out_type.append(jax.ShapeDtypeStruct((_NC * NKEYP, _CW), f32))

    return pl.kernel(
        body,
        out_type=out_type,
        mesh=mesh,
        compiler_params=pltpu.CompilerParams(use_tc_tiling_on_sc=False),
        scratch_types=[
            pltpu.VMEM_SHARED((NKEYP, _CW), f32),
            pltpu.VMEM((_NBI, _BB), jnp.int32),
            pltpu.VMEM((_NBI, _BB), jnp.int32),
            pltpu.VMEM((_NBI, _BB, _CW), f32),
            pltpu.VMEM((_BB, _CW), f32),
            pltpu.SemaphoreType.DMA,
        ],
    )


def _tc_combine(S, cnt, x, WrT_l, WtT_l, b_l, last):
    T, N, D = S.shape
    BN = 1000
    assert N % BN == 0

    def body(s_ref, c_ref, x_ref, wr_ref, wt_ref, b_ref, o_ref):
        cs = c_ref[...]
        cv = cs[0, :, :, 0:1] + cs[1, :, :, 0:1]
        inv = 1.0 / jnp.maximum(cv, 1.0)
        acc = jnp.dot(x_ref[...], wt_ref[...],
                      preferred_element_type=jnp.float32) + b_ref[...]
        for i in range(T):
            acc = acc + jnp.dot(s_ref[i] * inv[i], wr_ref[i],
                                preferred_element_type=jnp.float32)
        if last:
            m = jnp.max(acc, axis=-1, keepdims=True)
            ex = jnp.exp(acc - m)
            o_ref[...] = acc - m - jnp.log(jnp.sum(ex, axis=-1, keepdims=True))
        else:
            o_ref[...] = jnp.maximum(acc, 0.0)

    return pl.pallas_call(
        body,
        grid=(N // BN,),
        in_specs=[
            pl.BlockSpec((T, BN, D), lambda i: (0, i, 0)),
            pl.BlockSpec((_NC, T, BN, _CW), lambda i: (0, 0, i, 0)),
            pl.BlockSpec((BN, D), lambda i: (i, 0)),
            pl.BlockSpec((T, D, D), lambda i: (0, 0, 0)),
            pl.BlockSpec((D, D), lambda i: (0, 0)),
            pl.BlockSpec((1, D), lambda i: (0, 0)),
        ],
        out_specs=pl.BlockSpec((BN, D), lambda i: (i, 0)),
        out_shape=jax.ShapeDtypeStruct((N, D), jnp.float32),
    )(S, cnt, x, WrT_l, WtT_l, b_l.reshape(1, D))


def kernel(x_dict, edge_index, edge_type, node_type, local_node_idx,
           W_rel, W_root, b_root):
    N, D = x_dict.shape
    E = edge_index.shape[1]
    L, T = W_rel.shape[0], W_rel.shape[1]
    NCH = D // _CW
    NKEY = T * N
    NKEYP = _round_up(NKEY + 1, _NS * 8)
    E_pad = _round_up(E, _NS * _NC * _SB)

    src = edge_index[0]
    dst = edge_index[1]
    pad = E_pad - E
    key = edge_type * N + dst
    keyp = jnp.concatenate([key, jnp.full((pad,), NKEY, jnp.int32)])
    srcp = jnp.concatenate([src, jnp.zeros((pad,), jnp.int32)])
    key2 = keyp.reshape(E_pad // _BB, _BB)
    src8 = (srcp[None, :] + (jnp.arange(NCH, dtype=jnp.int32) * N)[:, None])
    src8 = src8.reshape(NCH * E_pad // _BB, _BB)
    zeros_h = jnp.zeros((NKEYP // _NS, _CW), jnp.float32)
    ones_h = jnp.ones((_BB, _CW), jnp.float32)

    WrT = W_rel.transpose(0, 1, 3, 2)
    WtT = W_root[:, 0].transpose(0, 2, 1)
    b = b_root[:, 0]

    def chunkify(h):
        return h.reshape(N, NCH, _CW).transpose(1, 0, 2).reshape(NCH * N, _CW)

    h = x_dict
    cnt = None
    for l in range(L):
        sc = _make_sc_segsum(N, E_pad, NCH, NKEY, NKEYP, l == 0)
        outs = sc(chunkify(h), src8, key2, zeros_h, ones_h)
        if l == 0:
            s_flat, c_flat = outs
            cnt = c_flat.reshape(_NC, NKEYP, _CW)[:, :NKEY].reshape(_NC, T, N, _CW)
        else:
            (s_flat,) = outs
        S = (s_flat.reshape(NCH, NKEYP, _CW)[:, :NKEY]
             .reshape(NCH, T, N, _CW).transpose(1, 2, 0, 3).reshape(T, N, D))
        h = _tc_combine(S, cnt, h, WrT[l], WtT[l], b[l], last=(l == L - 1))
    return h
